# Initial kernel scaffold; baseline (speedup 1.0000x reference)
import jax
import jax.numpy as jnp
from jax.experimental import pallas as pl


def kernel(x_categorical, x_numerical, tables, bn_num_g, bn_num_b, W1, b1, g1, be1, W2, b2, g2, be2, W3, b3, g3, be3, W4, b4):
    raise NotImplementedError("write your pallas kernel here")


# speedup vs baseline: 7.6863x; 7.6863x over previous
import functools

import jax
import jax.numpy as jnp
from jax import lax
from jax.experimental import pallas as pl
from jax.experimental.pallas import tpu as pltpu
from jax.experimental.pallas import tpu_sc as plsc

B = 16384
F = 26
V = 100000
D = 32
NUM = 13
EPS = 1e-5

ROWS = B * F
NW = 32
RPW = ROWS // NW
CHUNK = 128
KCH = 13
OUTER = RPW // (KCH * CHUNK)
NC = 2


def _sc_gather(tables2d, idx2d):
    mesh = plsc.VectorSubcoreMesh(core_axis_name="c", subcore_axis_name="s")

    @functools.partial(
        pl.kernel,
        mesh=mesh,
        out_type=jax.ShapeDtypeStruct((ROWS, D), jnp.float32),
        compiler_params=pltpu.CompilerParams(use_tc_tiling_on_sc=False),
        scratch_types=[
            pltpu.VMEM((RPW // CHUNK, CHUNK), jnp.int32),
            pltpu.VMEM((KCH * CHUNK, D), jnp.float32),
            pltpu.SemaphoreType.DMA,
        ],
    )
    def gk(tab_hbm, idx_hbm, out_hbm, idx_v, rows_v, sem):
        wid = lax.axis_index("s") * NC + lax.axis_index("c")
        row0 = wid * (RPW // CHUNK)
        pltpu.sync_copy(idx_hbm.at[pl.ds(row0, RPW // CHUNK)], idx_v)

        def outer(it, carry):
            copies = []
            for j in range(KCH):
                copies.append(pltpu.async_copy(
                    tab_hbm.at[idx_v.at[it * KCH + j]],
                    rows_v.at[pl.ds(j * CHUNK, CHUNK)],
                    sem))
            for cp in copies:
                cp.wait()
            base = wid * RPW + it * (KCH * CHUNK)
            pltpu.sync_copy(rows_v, out_hbm.at[pl.ds(base, KCH * CHUNK)])
            return carry

        lax.fori_loop(0, OUTER, outer, 0)

    return gk(tables2d, idx2d)


def _num_bn_coeffs(x_num, g, b):
    def body(xn_ref, g_ref, b_ref, a_ref, c_ref):
        x = xn_ref[...]
        mu = jnp.mean(x, axis=0, keepdims=True)
        var = jnp.mean((x - mu) * (x - mu), axis=0, keepdims=True)
        a = g_ref[...] * lax.rsqrt(var + EPS)
        a_ref[...] = a
        c_ref[...] = b_ref[...] - mu * a

    return pl.pallas_call(
        body,
        out_shape=[jax.ShapeDtypeStruct((1, NUM), jnp.float32),
                   jax.ShapeDtypeStruct((1, NUM), jnp.float32)],
    )(x_num, g.reshape(1, NUM), b.reshape(1, NUM))


_TB = 1024


def _layer1(emb, x_num, a_n, c_n, w1a, w1b, b1):
    def body(e_ref, xn_ref, an_ref, cn_ref, wa_ref, wb_ref, b_ref,
             h_ref, st_ref):
        i = pl.program_id(0)
        xn = xn_ref[...] * an_ref[...] + cn_ref[...]
        h = jnp.dot(e_ref[...], wa_ref[...],
                    preferred_element_type=jnp.float32)
        h = h + jnp.dot(xn, wb_ref[...], preferred_element_type=jnp.float32)
        h = jnp.maximum(h + b_ref[...], 0.0)
        h_ref[...] = h
        st = jnp.concatenate([jnp.sum(h, axis=0, keepdims=True),
                              jnp.sum(h * h, axis=0, keepdims=True)], axis=0)

        @pl.when(i == 0)
        def _():
            st_ref[...] = st

        @pl.when(i > 0)
        def _():
            st_ref[...] = st_ref[...] + st

    dout = w1a.shape[1]
    return pl.pallas_call(
        body,
        grid=(B // _TB,),
        in_specs=[
            pl.BlockSpec((_TB, F * D), lambda i: (i, 0)),
            pl.BlockSpec((_TB, NUM), lambda i: (i, 0)),
            pl.BlockSpec((1, NUM), lambda i: (0, 0)),
            pl.BlockSpec((1, NUM), lambda i: (0, 0)),
            pl.BlockSpec((F * D, dout), lambda i: (0, 0)),
            pl.BlockSpec((NUM, dout), lambda i: (0, 0)),
            pl.BlockSpec((1, dout), lambda i: (0, 0)),
        ],
        out_specs=[
            pl.BlockSpec((_TB, dout), lambda i: (i, 0)),
            pl.BlockSpec((2, dout), lambda i: (0, 0)),
        ],
        out_shape=[jax.ShapeDtypeStruct((B, dout), jnp.float32),
                   jax.ShapeDtypeStruct((2, dout), jnp.float32)],
    )(emb, x_num, a_n, c_n, w1a, w1b, b1)


def _mid_layer(h_prev, a, c, w, b):
    def body(hp_ref, a_ref, c_ref, w_ref, b_ref, h_ref, st_ref):
        i = pl.program_id(0)
        x = hp_ref[...] * a_ref[...] + c_ref[...]
        h = jnp.dot(x, w_ref[...], preferred_element_type=jnp.float32)
        h = jnp.maximum(h + b_ref[...], 0.0)
        h_ref[...] = h
        st = jnp.concatenate([jnp.sum(h, axis=0, keepdims=True),
                              jnp.sum(h * h, axis=0, keepdims=True)], axis=0)

        @pl.when(i == 0)
        def _():
            st_ref[...] = st

        @pl.when(i > 0)
        def _():
            st_ref[...] = st_ref[...] + st

    din, dout = w.shape
    return pl.pallas_call(
        body,
        grid=(B // _TB,),
        in_specs=[
            pl.BlockSpec((_TB, din), lambda i: (i, 0)),
            pl.BlockSpec((1, din), lambda i: (0, 0)),
            pl.BlockSpec((1, din), lambda i: (0, 0)),
            pl.BlockSpec((din, dout), lambda i: (0, 0)),
            pl.BlockSpec((1, dout), lambda i: (0, 0)),
        ],
        out_specs=[
            pl.BlockSpec((_TB, dout), lambda i: (i, 0)),
            pl.BlockSpec((2, dout), lambda i: (0, 0)),
        ],
        out_shape=[jax.ShapeDtypeStruct((B, dout), jnp.float32),
                   jax.ShapeDtypeStruct((2, dout), jnp.float32)],
    )(h_prev, a, c, w, b)


def _final_layer(h_prev, a, c, w, b):
    def body(hp_ref, a_ref, c_ref, w_ref, b_ref, o_ref):
        x = hp_ref[...] * a_ref[...] + c_ref[...]
        o_ref[...] = jnp.dot(x, w_ref[...],
                             preferred_element_type=jnp.float32) + b_ref[...]

    din, dout = w.shape
    return pl.pallas_call(
        body,
        grid=(B // _TB,),
        in_specs=[
            pl.BlockSpec((_TB, din), lambda i: (i, 0)),
            pl.BlockSpec((1, din), lambda i: (0, 0)),
            pl.BlockSpec((1, din), lambda i: (0, 0)),
            pl.BlockSpec((din, dout), lambda i: (0, 0)),
            pl.BlockSpec((1, dout), lambda i: (0, 0)),
        ],
        out_specs=pl.BlockSpec((_TB, dout), lambda i: (i, 0)),
        out_shape=jax.ShapeDtypeStruct((B, dout), jnp.float32),
    )(h_prev, a, c, w, b)


def _bn_coeffs(stats, g, be):
    s, s2 = stats[0:1], stats[1:2]
    mu = s / B
    var = s2 / B - mu * mu
    a = g.reshape(1, -1) * lax.rsqrt(var + EPS)
    c = be.reshape(1, -1) - mu * a
    return a, c


def kernel(x_categorical, x_numerical, tables, bn_num_g, bn_num_b,
           W1, b1, g1, be1, W2, b2, g2, be2, W3, b3, g3, be3, W4, b4):
    offs = (jnp.arange(F, dtype=jnp.int32) * V)[None, :]
    flat_idx = (x_categorical.astype(jnp.int32) + offs).reshape(ROWS // CHUNK,
                                                                CHUNK)
    tables2d = tables.reshape(F * V, D)

    emb = _sc_gather(tables2d, flat_idx).reshape(B, F * D)

    a_n, c_n = _num_bn_coeffs(x_numerical, bn_num_g, bn_num_b)

    w1a, w1b = W1[:F * D], W1[F * D:]
    h1, st1 = _layer1(emb, x_numerical, a_n, c_n, w1a, w1b, b1.reshape(1, -1))
    a1, c1 = _bn_coeffs(st1, g1, be1)
    h2, st2 = _mid_layer(h1, a1, c1, W2, b2.reshape(1, -1))
    a2, c2 = _bn_coeffs(st2, g2, be2)
    h3, st3 = _mid_layer(h2, a2, c2, W3, b3.reshape(1, -1))
    a3, c3 = _bn_coeffs(st3, g3, be3)
    return _final_layer(h3, a3, c3, W4, b4.reshape(1, -1))


# speedup vs baseline: 7.6987x; 1.0016x over previous
import functools

import jax
import jax.numpy as jnp
from jax import lax
from jax.experimental import pallas as pl
from jax.experimental.pallas import tpu as pltpu
from jax.experimental.pallas import tpu_sc as plsc

B = 16384
F = 26
V = 100000
D = 32
NUM = 13
EPS = 1e-5

ROWS = B * F
NW = 32
RPW = ROWS // NW
CHUNK = 128
KCH = 13
OUTER = RPW // (KCH * CHUNK)
NC = 2


def _sc_gather(tables2d, idx2d):
    mesh = plsc.VectorSubcoreMesh(core_axis_name="c", subcore_axis_name="s")

    @functools.partial(
        pl.kernel,
        mesh=mesh,
        out_type=jax.ShapeDtypeStruct((ROWS, D), jnp.float32),
        compiler_params=pltpu.CompilerParams(use_tc_tiling_on_sc=False),
        scratch_types=[
            pltpu.VMEM((RPW // CHUNK, CHUNK), jnp.int32),
            pltpu.VMEM((KCH * CHUNK, D), jnp.float32),
            pltpu.SemaphoreType.DMA,
        ],
    )
    def gk(tab_hbm, idx_hbm, out_hbm, idx_v, rows_v, sem):
        wid = lax.axis_index("s") * NC + lax.axis_index("c")
        row0 = wid * (RPW // CHUNK)
        pltpu.sync_copy(idx_hbm.at[pl.ds(row0, RPW // CHUNK)], idx_v)

        def outer(it, carry):
            copies = []
            for j in range(KCH):
                copies.append(pltpu.async_copy(
                    tab_hbm.at[idx_v.at[it * KCH + j]],
                    rows_v.at[pl.ds(j * CHUNK, CHUNK)],
                    sem))
            for cp in copies:
                cp.wait()
            base = wid * RPW + it * (KCH * CHUNK)
            pltpu.sync_copy(rows_v, out_hbm.at[pl.ds(base, KCH * CHUNK)])
            return carry

        lax.fori_loop(0, OUTER, outer, 0)

    return gk(tables2d, idx2d)


def _num_bn_coeffs(x_num, g, b):
    def body(xn_ref, g_ref, b_ref, a_ref, c_ref):
        x = xn_ref[...]
        mu = jnp.mean(x, axis=0, keepdims=True)
        var = jnp.mean((x - mu) * (x - mu), axis=0, keepdims=True)
        a = g_ref[...] * lax.rsqrt(var + EPS)
        a_ref[...] = a
        c_ref[...] = b_ref[...] - mu * a

    return pl.pallas_call(
        body,
        out_shape=[jax.ShapeDtypeStruct((1, NUM), jnp.float32),
                   jax.ShapeDtypeStruct((1, NUM), jnp.float32)],
    )(x_num, g.reshape(1, NUM), b.reshape(1, NUM))


_TB = 1024


def _layer1(emb, x_num, a_n, c_n, w1a, w1b, b1):
    def body(e_ref, xn_ref, an_ref, cn_ref, wa_ref, wb_ref, b_ref,
             h_ref, st_ref):
        i = pl.program_id(0)
        xn = xn_ref[...] * an_ref[...] + cn_ref[...]
        h = jnp.dot(e_ref[...], wa_ref[...],
                    preferred_element_type=jnp.float32)
        h = h + jnp.dot(xn, wb_ref[...], preferred_element_type=jnp.float32)
        h = jnp.maximum(h + b_ref[...], 0.0)
        h_ref[...] = h
        st = jnp.concatenate([jnp.sum(h, axis=0, keepdims=True),
                              jnp.sum(h * h, axis=0, keepdims=True)], axis=0)

        @pl.when(i == 0)
        def _():
            st_ref[...] = st

        @pl.when(i > 0)
        def _():
            st_ref[...] = st_ref[...] + st

    dout = w1a.shape[1]
    return pl.pallas_call(
        body,
        grid=(B // _TB,),
        in_specs=[
            pl.BlockSpec((_TB, F * D), lambda i: (i, 0)),
            pl.BlockSpec((_TB, NUM), lambda i: (i, 0)),
            pl.BlockSpec((1, NUM), lambda i: (0, 0)),
            pl.BlockSpec((1, NUM), lambda i: (0, 0)),
            pl.BlockSpec((F * D, dout), lambda i: (0, 0)),
            pl.BlockSpec((NUM, dout), lambda i: (0, 0)),
            pl.BlockSpec((1, dout), lambda i: (0, 0)),
        ],
        out_specs=[
            pl.BlockSpec((_TB, dout), lambda i: (i, 0)),
            pl.BlockSpec((2, dout), lambda i: (0, 0)),
        ],
        out_shape=[jax.ShapeDtypeStruct((B, dout), jnp.float32),
                   jax.ShapeDtypeStruct((2, dout), jnp.float32)],
    )(emb, x_num, a_n, c_n, w1a, w1b, b1)


def _mid_layer(h_prev, a, c, w, b):
    def body(hp_ref, a_ref, c_ref, w_ref, b_ref, h_ref, st_ref):
        i = pl.program_id(0)
        x = hp_ref[...] * a_ref[...] + c_ref[...]
        h = jnp.dot(x, w_ref[...], preferred_element_type=jnp.float32)
        h = jnp.maximum(h + b_ref[...], 0.0)
        h_ref[...] = h
        st = jnp.concatenate([jnp.sum(h, axis=0, keepdims=True),
                              jnp.sum(h * h, axis=0, keepdims=True)], axis=0)

        @pl.when(i == 0)
        def _():
            st_ref[...] = st

        @pl.when(i > 0)
        def _():
            st_ref[...] = st_ref[...] + st

    din, dout = w.shape
    return pl.pallas_call(
        body,
        grid=(B // _TB,),
        in_specs=[
            pl.BlockSpec((_TB, din), lambda i: (i, 0)),
            pl.BlockSpec((1, din), lambda i: (0, 0)),
            pl.BlockSpec((1, din), lambda i: (0, 0)),
            pl.BlockSpec((din, dout), lambda i: (0, 0)),
            pl.BlockSpec((1, dout), lambda i: (0, 0)),
        ],
        out_specs=[
            pl.BlockSpec((_TB, dout), lambda i: (i, 0)),
            pl.BlockSpec((2, dout), lambda i: (0, 0)),
        ],
        out_shape=[jax.ShapeDtypeStruct((B, dout), jnp.float32),
                   jax.ShapeDtypeStruct((2, dout), jnp.float32)],
    )(h_prev, a, c, w, b)


def _final_layer(h_prev, a, c, w, b):
    def body(hp_ref, a_ref, c_ref, w_ref, b_ref, o_ref):
        x = hp_ref[...] * a_ref[...] + c_ref[...]
        o_ref[...] = jnp.dot(x, w_ref[...],
                             preferred_element_type=jnp.float32) + b_ref[...]

    din, dout = w.shape
    return pl.pallas_call(
        body,
        grid=(B // _TB,),
        in_specs=[
            pl.BlockSpec((_TB, din), lambda i: (i, 0)),
            pl.BlockSpec((1, din), lambda i: (0, 0)),
            pl.BlockSpec((1, din), lambda i: (0, 0)),
            pl.BlockSpec((din, dout), lambda i: (0, 0)),
            pl.BlockSpec((1, dout), lambda i: (0, 0)),
        ],
        out_specs=pl.BlockSpec((_TB, dout), lambda i: (i, 0)),
        out_shape=jax.ShapeDtypeStruct((B, dout), jnp.float32),
    )(h_prev, a, c, w, b)


def _bn_coeffs(stats, g, be):
    s, s2 = stats[0:1], stats[1:2]
    mu = s / B
    var = s2 / B - mu * mu
    a = g.reshape(1, -1) * lax.rsqrt(var + EPS)
    c = be.reshape(1, -1) - mu * a
    return a, c


def kernel(x_categorical, x_numerical, tables, bn_num_g, bn_num_b,
           W1, b1, g1, be1, W2, b2, g2, be2, W3, b3, g3, be3, W4, b4):
    offs = (jnp.arange(F, dtype=jnp.int32) * V)[None, :]
    flat_idx = (x_categorical.astype(jnp.int32) + offs).reshape(ROWS // CHUNK,
                                                                CHUNK)
    t128 = jax.lax.optimization_barrier(tables.reshape(F * V // 4, 4 * D))
    tables2d = t128.reshape(F * V, D)

    emb = _sc_gather(tables2d, flat_idx).reshape(B, F * D)

    a_n, c_n = _num_bn_coeffs(x_numerical, bn_num_g, bn_num_b)

    w1a, w1b = W1[:F * D], W1[F * D:]
    h1, st1 = _layer1(emb, x_numerical, a_n, c_n, w1a, w1b, b1.reshape(1, -1))
    a1, c1 = _bn_coeffs(st1, g1, be1)
    h2, st2 = _mid_layer(h1, a1, c1, W2, b2.reshape(1, -1))
    a2, c2 = _bn_coeffs(st2, g2, be2)
    h3, st3 = _mid_layer(h2, a2, c2, W3, b3.reshape(1, -1))
    a3, c3 = _bn_coeffs(st3, g3, be3)
    return _final_layer(h3, a3, c3, W4, b4.reshape(1, -1))
